# Initial kernel scaffold; baseline (speedup 1.0000x reference)
import jax
import jax.numpy as jnp
from jax.experimental import pallas as pl


def kernel(x, edge_index, W, b):
    raise NotImplementedError("write your pallas kernel here")


# speedup vs baseline: 17.3926x; 17.3926x over previous
import functools

import jax
import jax.numpy as jnp
from jax import lax
from jax.experimental import pallas as pl
from jax.experimental.pallas import tpu as pltpu
from jax.experimental.pallas import tpu_sc as plsc

N, E, D = 10000, 320000, 128
N2 = 10240
NC, NS = 2, 16
NW = NC * NS
EPW = E // NW
CH = 80
NCHUNK = EPW // CH
RPT = N2 // NS
BLK = 1024
NBLK = N2 // BLK

_MESH = plsc.VectorSubcoreMesh(core_axis_name="c", subcore_axis_name="s")


@functools.partial(
    pl.kernel,
    out_type=jax.ShapeDtypeStruct((NC * N2, 1), jnp.float32),
    mesh=_MESH,
    scratch_types=[
        pltpu.VMEM((CH,), jnp.int32),
        pltpu.VMEM((CH, 1), jnp.float32),
        pltpu.VMEM_SHARED((N2, 1), jnp.float32),
    ],
)
def _sc_hist(dst_hbm, ones_hbm, zeros_hbm, deg_hbm, idx_v, ones_v, acc):
    c = lax.axis_index("c")
    s = lax.axis_index("s")
    wid = c * NS + s
    r0 = s * RPT
    pltpu.sync_copy(zeros_hbm.at[pl.ds(r0, RPT)], acc.at[pl.ds(r0, RPT)])
    pltpu.sync_copy(ones_hbm, ones_v)
    plsc.subcore_barrier()
    base = wid * EPW

    def body(i, carry):
        off = pl.multiple_of(base + i * CH, 8)
        pltpu.sync_copy(dst_hbm.at[pl.ds(off, CH)], idx_v)
        pltpu.sync_copy(ones_v, acc.at[idx_v], add=True)
        return carry

    lax.fori_loop(0, NCHUNK, body, 0)
    plsc.subcore_barrier()
    pltpu.sync_copy(acc.at[pl.ds(r0, RPT)],
                    deg_hbm.at[pl.ds(c * N2 + r0, RPT)])


@functools.partial(
    pl.kernel,
    out_type=jax.ShapeDtypeStruct((NC * N2, D), jnp.float32),
    mesh=_MESH,
    scratch_types=[
        pltpu.VMEM((CH,), jnp.int32),
        pltpu.VMEM((CH,), jnp.int32),
        pltpu.VMEM((CH, D), jnp.float32),
        pltpu.VMEM_SHARED((N2, D), jnp.float32),
        pltpu.SemaphoreType.DMA,
    ],
)
def _sc_msg(g_hbm, src_hbm, dst_hbm, zeros_hbm, q_hbm,
            idxs_v, idxd_v, rows_v, acc, sem):
    c = lax.axis_index("c")
    s = lax.axis_index("s")
    wid = c * NS + s
    r0 = s * RPT
    pltpu.sync_copy(zeros_hbm.at[pl.ds(r0, RPT)], acc.at[pl.ds(r0, RPT)])
    plsc.subcore_barrier()
    base = wid * EPW

    def body(i, carry):
        off = pl.multiple_of(base + i * CH, 8)
        pltpu.sync_copy(src_hbm.at[pl.ds(off, CH)], idxs_v)
        pltpu.sync_copy(dst_hbm.at[pl.ds(off, CH)], idxd_v)
        pltpu.async_copy(g_hbm.at[idxs_v], rows_v, sem).wait()
        pltpu.sync_copy(rows_v, acc.at[idxd_v], add=True)
        return carry

    lax.fori_loop(0, NCHUNK, body, 0)
    plsc.subcore_barrier()
    pltpu.sync_copy(acc.at[pl.ds(r0, RPT)],
                    q_hbm.at[pl.ds(c * N2 + r0, RPT)])


def _tc_scale_body(x_ref, w_ref, degp_ref, g_ref, dinv_ref):
    deg = 1.0 + degp_ref[0] + degp_ref[1]
    di = lax.rsqrt(deg)
    h = jnp.dot(x_ref[...], w_ref[...], preferred_element_type=jnp.float32)
    g_ref[...] = h * di
    dinv_ref[...] = di


def _tc_scale(x2, W, degp):
    return pl.pallas_call(
        _tc_scale_body,
        grid=(NBLK,),
        in_specs=[
            pl.BlockSpec((BLK, D), lambda i: (i, 0)),
            pl.BlockSpec((D, D), lambda i: (0, 0)),
            pl.BlockSpec((NC, BLK, 1), lambda i: (0, i, 0)),
        ],
        out_specs=[
            pl.BlockSpec((BLK, D), lambda i: (i, 0)),
            pl.BlockSpec((BLK, 1), lambda i: (i, 0)),
        ],
        out_shape=[
            jax.ShapeDtypeStruct((N2, D), jnp.float32),
            jax.ShapeDtypeStruct((N2, 1), jnp.float32),
        ],
    )(x2, W, degp)


def _tc_combine_body(q0_ref, q1_ref, g_ref, dinv_ref, b_ref, o_ref):
    acc = q0_ref[...] + q1_ref[...] + g_ref[...]
    o_ref[...] = jnp.maximum(acc * dinv_ref[...] + b_ref[...], 0.0)


def _tc_combine(qf, g, dinv, b2):
    return pl.pallas_call(
        _tc_combine_body,
        grid=(NBLK,),
        in_specs=[
            pl.BlockSpec((BLK, D), lambda i: (i, 0)),
            pl.BlockSpec((BLK, D), lambda i: (i + NBLK, 0)),
            pl.BlockSpec((BLK, D), lambda i: (i, 0)),
            pl.BlockSpec((BLK, 1), lambda i: (i, 0)),
            pl.BlockSpec((1, D), lambda i: (0, 0)),
        ],
        out_specs=pl.BlockSpec((BLK, D), lambda i: (i, 0)),
        out_shape=jax.ShapeDtypeStruct((N2, D), jnp.float32),
    )(qf, qf, g, dinv, b2)


def kernel(x, edge_index, W, b):
    src = edge_index[0]
    dst = edge_index[1]
    x2 = jnp.pad(x, ((0, N2 - N), (0, 0)))
    ones_h = jnp.ones((CH, 1), jnp.float32)
    z1 = jnp.zeros((N2, 1), jnp.float32)
    zD = jnp.zeros((N2, D), jnp.float32)

    degp = _sc_hist(dst, ones_h, z1).reshape(NC, N2, 1)
    g, dinv = _tc_scale(x2, W, degp)
    qf = _sc_msg(g, src, dst, zD)
    out = _tc_combine(qf, g, dinv, b.reshape(1, D))
    return out[:N]


# speedup vs baseline: 26.5923x; 1.5289x over previous
import functools

import jax
import jax.numpy as jnp
from jax import lax
from jax.experimental import pallas as pl
from jax.experimental.pallas import tpu as pltpu
from jax.experimental.pallas import tpu_sc as plsc

N, E, D = 10000, 320000, 128
N2 = 10240
NC, NS = 2, 16
NW = NC * NS
EPW = E // NW
CH = 80
NCHUNK = EPW // CH
RPT = N2 // NS
BLK = 1024
NBLK = N2 // BLK

_MESH = plsc.VectorSubcoreMesh(core_axis_name="c", subcore_axis_name="s")


@functools.partial(
    pl.kernel,
    out_type=jax.ShapeDtypeStruct((NC * N2,), jnp.float32),
    mesh=_MESH,
    scratch_types=[
        pltpu.VMEM((CH,), jnp.int32),
        pltpu.VMEM((CH,), jnp.float32),
        pltpu.VMEM_SHARED((N2,), jnp.float32),
    ],
)
def _sc_hist(dst_hbm, zeros_hbm, deg_hbm, idx_v, ones_v, acc):
    c = lax.axis_index("c")
    s = lax.axis_index("s")
    wid = c * NS + s
    r0 = s * RPT
    pltpu.sync_copy(zeros_hbm.at[pl.ds(r0, RPT)], acc.at[pl.ds(r0, RPT)])
    for k in range(CH // 16):
        ones_v[pl.ds(k * 16, 16)] = jnp.full((16,), 1.0, jnp.float32)
    plsc.subcore_barrier()
    base = wid * EPW

    def body(i, carry):
        off = pl.multiple_of(base + i * CH, 8)
        pltpu.sync_copy(dst_hbm.at[pl.ds(off, CH)], idx_v)
        pltpu.sync_copy(ones_v, acc.at[idx_v], add=True)
        return carry

    lax.fori_loop(0, NCHUNK, body, 0)
    plsc.subcore_barrier()
    pltpu.sync_copy(acc.at[pl.ds(r0, RPT)],
                    deg_hbm.at[pl.ds(c * N2 + r0, RPT)])


@functools.partial(
    pl.kernel,
    out_type=jax.ShapeDtypeStruct((NC * N2, D), jnp.float32),
    mesh=_MESH,
    scratch_types=[
        pltpu.VMEM((CH,), jnp.int32),
        pltpu.VMEM((CH,), jnp.int32),
        pltpu.VMEM((CH,), jnp.int32),
        pltpu.VMEM((CH,), jnp.int32),
        pltpu.VMEM((CH, D), jnp.float32),
        pltpu.VMEM((CH, D), jnp.float32),
        pltpu.VMEM_SHARED((N2, D), jnp.float32),
    ] + [pltpu.SemaphoreType.DMA] * 6,
)
def _sc_msg(g_hbm, srcf_hbm, dstf_hbm, zeros_hbm, q_hbm,
            idxs0, idxs1, idxd0, idxd1, rows0, rows1, acc,
            gsem0, gsem1, isem0, isem1, dsem0, dsem1):
    c = lax.axis_index("c")
    s = lax.axis_index("s")
    wid = c * NS + s
    r0 = s * RPT
    pltpu.sync_copy(zeros_hbm.at[pl.ds(r0, RPT)], acc.at[pl.ds(r0, RPT)])
    plsc.subcore_barrier()

    rows = (rows0, rows1)
    idxs = (idxs0, idxs1)
    idxd = (idxd0, idxd1)
    gsem = (gsem0, gsem1)
    isem = (isem0, isem1)
    dsem = (dsem0, dsem1)
    base = wid * EPW

    def eoff(i):
        return pl.multiple_of(base + i * CH, 8)

    pltpu.sync_copy(srcf_hbm.at[pl.ds(eoff(0), CH)], idxs0)
    pltpu.async_copy(dstf_hbm.at[pl.ds(eoff(0), CH)], idxd0, dsem0)
    pltpu.async_copy(g_hbm.at[idxs0], rows0, gsem0)
    pltpu.async_copy(srcf_hbm.at[pl.ds(eoff(1), CH)], idxs1, isem1)

    def step(i, p):
        pltpu.make_async_copy(g_hbm.at[idxs[p]], rows[p], gsem[p]).wait()

        @pl.when(i + 2 < NCHUNK)
        def _():
            pltpu.async_copy(srcf_hbm.at[pl.ds(eoff(i + 2), CH)],
                             idxs[p], isem[p])

        @pl.when(i + 1 < NCHUNK)
        def _():
            pltpu.make_async_copy(srcf_hbm.at[pl.ds(eoff(i + 1), CH)],
                                  idxs[1 - p], isem[1 - p]).wait()
            pltpu.async_copy(g_hbm.at[idxs[1 - p]], rows[1 - p], gsem[1 - p])
            pltpu.async_copy(dstf_hbm.at[pl.ds(eoff(i + 1), CH)],
                             idxd[1 - p], dsem[1 - p])

        pltpu.make_async_copy(dstf_hbm.at[pl.ds(eoff(i), CH)],
                              idxd[p], dsem[p]).wait()
        pltpu.sync_copy(rows[p], acc.at[idxd[p]], add=True)

    def body(j, carry):
        step(2 * j, 0)
        step(2 * j + 1, 1)
        return carry

    lax.fori_loop(0, NCHUNK // 2, body, 0)
    step(NCHUNK - 1, 0)
    plsc.subcore_barrier()
    pltpu.sync_copy(acc.at[pl.ds(r0, RPT)],
                    q_hbm.at[pl.ds(c * N2 + r0, RPT)])


def _tc_scale_body(x_ref, w_ref, degp_ref, g_ref, dinv_ref):
    deg = 1.0 + degp_ref[0] + degp_ref[1]
    di = lax.rsqrt(deg)
    h = jnp.dot(x_ref[...], w_ref[...], preferred_element_type=jnp.float32)
    g_ref[...] = h * di
    dinv_ref[...] = di


def _tc_scale(x2, W, degp):
    return pl.pallas_call(
        _tc_scale_body,
        grid=(NBLK,),
        in_specs=[
            pl.BlockSpec((BLK, D), lambda i: (i, 0)),
            pl.BlockSpec((D, D), lambda i: (0, 0)),
            pl.BlockSpec((NC, BLK, 1), lambda i: (0, i, 0)),
        ],
        out_specs=[
            pl.BlockSpec((BLK, D), lambda i: (i, 0)),
            pl.BlockSpec((BLK, 1), lambda i: (i, 0)),
        ],
        out_shape=[
            jax.ShapeDtypeStruct((N2, D), jnp.float32),
            jax.ShapeDtypeStruct((N2, 1), jnp.float32),
        ],
    )(x2, W, degp)


def _tc_combine_body(q0_ref, q1_ref, g_ref, dinv_ref, b_ref, o_ref):
    acc = q0_ref[...] + q1_ref[...] + g_ref[...]
    o_ref[...] = jnp.maximum(acc * dinv_ref[...] + b_ref[...], 0.0)


def _tc_combine(qf, g, dinv, b2):
    return pl.pallas_call(
        _tc_combine_body,
        grid=(NBLK,),
        in_specs=[
            pl.BlockSpec((BLK, D), lambda i: (i, 0)),
            pl.BlockSpec((BLK, D), lambda i: (i + NBLK, 0)),
            pl.BlockSpec((BLK, D), lambda i: (i, 0)),
            pl.BlockSpec((BLK, 1), lambda i: (i, 0)),
            pl.BlockSpec((1, D), lambda i: (0, 0)),
        ],
        out_specs=pl.BlockSpec((BLK, D), lambda i: (i, 0)),
        out_shape=jax.ShapeDtypeStruct((N2, D), jnp.float32),
    )(qf, qf, g, dinv, b2)


def kernel(x, edge_index, W, b):
    x2 = jnp.pad(x, ((0, N2 - N), (0, 0)))
    z1 = jnp.zeros((N2,), jnp.float32)
    zD = jnp.zeros((N2, D), jnp.float32)

    degp = _sc_hist(edge_index[1], z1).reshape(NC, N2, 1)
    g, dinv = _tc_scale(x2, W, degp)
    qf = _sc_msg(g, edge_index[0], edge_index[1], zD)
    out = _tc_combine(qf, g, dinv, b.reshape(1, D))
    return out[:N]
